# baseline (speedup 1.0000x reference)
import functools

import jax
import jax.numpy as jnp
from jax import lax
from jax.experimental import pallas as pl
from jax.experimental.pallas import tpu as pltpu
from jax.experimental.pallas import tpu_sc as plsc

N = 10000
E = 160000
D = 16
H = 128
DD = D * D

NC = 2
NS = 16
NW = NC * NS
EPW = E // NW
CH = 125
NCH = EPW // CH
NP = 10240
RPS = NP // NS

EB = 2000
_MESH = plsc.VectorSubcoreMesh(core_axis_name="c", subcore_axis_name="s")
_SC_PARAMS = pltpu.CompilerParams(use_tc_tiling_on_sc=False)


def _fill_rows(ref, n, value):
    def body(i, carry):
        ref[i, :] = jnp.full((D,), value, jnp.float32)
        return carry
    lax.fori_loop(0, n, body, 0)


def _fire_drain(n, fire):
    def fire_body(j, carry):
        fire(j)
        return carry
    lax.fori_loop(0, n, fire_body, 0)

    def drain_body(j, carry):
        fire(0, wait=True)
        return carry
    lax.fori_loop(0, n, drain_body, 0)


@functools.partial(
    pl.kernel,
    out_type=(
        jax.ShapeDtypeStruct((E, D), jnp.float32),
        jax.ShapeDtypeStruct((NC, NP, D), jnp.float32),
    ),
    scratch_types=[
        pltpu.VMEM((NCH, CH), jnp.int32),
        pltpu.VMEM((NCH, CH), jnp.int32),
        pltpu.VMEM((EPW, D), jnp.float32),
        pltpu.VMEM((CH, D), jnp.float32),
        pltpu.VMEM((RPS, D), jnp.float32),
        pltpu.SemaphoreType.DMA,
        pltpu.SemaphoreType.DMA,
        pltpu.VMEM_SHARED((NP, D), jnp.float32),
    ],
    mesh=_MESH,
    compiler_params=_SC_PARAMS,
)
def _sc_gather_counts(x_hbm, src_hbm, dst_hbm, xs_out, cnt_out,
                      sidx, didx, rows, obuf, zbuf, gsem, csem, cnt_sh):
    c = lax.axis_index("c")
    s = lax.axis_index("s")
    wid = s * NC + c
    pltpu.sync_copy(src_hbm.at[pl.ds(wid * NCH, NCH)], sidx)
    pltpu.sync_copy(dst_hbm.at[pl.ds(wid * NCH, NCH)], didx)
    _fill_rows(obuf, CH, 1.0)
    _fill_rows(zbuf, RPS, 0.0)
    pltpu.sync_copy(zbuf, cnt_sh.at[pl.ds(s * RPS, RPS)])
    plsc.subcore_barrier()

    def gath(j, wait=False):
        d = pltpu.make_async_copy(x_hbm.at[sidx.at[j]],
                                  rows.at[pl.ds(j * CH, CH)], gsem)
        d.wait() if wait else d.start()
    _fire_drain(NCH, gath)

    def cadd(j, wait=False):
        d = pltpu.make_async_copy(obuf, cnt_sh.at[didx.at[j]], csem)
        d.wait() if wait else d.start(add=True)
    _fire_drain(NCH, cadd)

    pltpu.sync_copy(rows, xs_out.at[pl.ds(wid * EPW, EPW)])
    plsc.subcore_barrier()
    pltpu.sync_copy(cnt_sh.at[pl.ds(s * RPS, RPS)],
                    cnt_out.at[c].at[pl.ds(s * RPS, RPS)])


@functools.partial(
    pl.kernel,
    out_type=jax.ShapeDtypeStruct((E, D), jnp.float32),
    scratch_types=[
        pltpu.VMEM((NCH, CH), jnp.int32),
        pltpu.VMEM((EPW, D), jnp.float32),
        pltpu.SemaphoreType.DMA,
    ],
    mesh=_MESH,
    compiler_params=_SC_PARAMS,
)
def _sc_gather(x_hbm, src_hbm, xs_out, sidx, rows, gsem):
    c = lax.axis_index("c")
    s = lax.axis_index("s")
    wid = s * NC + c
    pltpu.sync_copy(src_hbm.at[pl.ds(wid * NCH, NCH)], sidx)

    def gath(j, wait=False):
        d = pltpu.make_async_copy(x_hbm.at[sidx.at[j]],
                                  rows.at[pl.ds(j * CH, CH)], gsem)
        d.wait() if wait else d.start()
    _fire_drain(NCH, gath)

    pltpu.sync_copy(rows, xs_out.at[pl.ds(wid * EPW, EPW)])


@functools.partial(
    pl.kernel,
    out_type=jax.ShapeDtypeStruct((NC, NP, D), jnp.float32),
    scratch_types=[
        pltpu.VMEM((NCH, CH), jnp.int32),
        pltpu.VMEM((EPW, D), jnp.float32),
        pltpu.VMEM((RPS, D), jnp.float32),
        pltpu.SemaphoreType.DMA,
        pltpu.VMEM_SHARED((NP, D), jnp.float32),
    ],
    mesh=_MESH,
    compiler_params=_SC_PARAMS,
)
def _sc_scatter(dst_hbm, msg_hbm, agg_out, didx, buf, zbuf, asem, agg_sh):
    c = lax.axis_index("c")
    s = lax.axis_index("s")
    wid = s * NC + c
    pltpu.sync_copy(dst_hbm.at[pl.ds(wid * NCH, NCH)], didx)
    pltpu.sync_copy(msg_hbm.at[pl.ds(wid * EPW, EPW)], buf)
    _fill_rows(zbuf, RPS, 0.0)
    pltpu.sync_copy(zbuf, agg_sh.at[pl.ds(s * RPS, RPS)])
    plsc.subcore_barrier()

    def sadd(j, wait=False):
        d = pltpu.make_async_copy(buf.at[pl.ds(j * CH, CH)],
                                  agg_sh.at[didx.at[j]], asem)
        d.wait() if wait else d.start(add=True)
    _fire_drain(NCH, sadd)

    plsc.subcore_barrier()
    pltpu.sync_copy(agg_sh.at[pl.ds(s * RPS, RPS)],
                    agg_out.at[c].at[pl.ds(s * RPS, RPS)])


def _bn_affine(x, gamma, beta):
    mu = jnp.mean(x, axis=0, keepdims=True)
    var = jnp.mean((x - mu) ** 2, axis=0, keepdims=True)
    a = gamma / jnp.sqrt(var + 1e-5)
    return jnp.concatenate([a, beta - mu * a], axis=0)


def _dense_body(e_ref, xs_ref, nodes_ref, gamma_ref, beta_ref,
                w1_ref, b1_ref, w2_ref, b2_ref, w3_ref, b3_ref,
                r_ref, s_ref, msg_ref, ac_s):
    @pl.when(pl.program_id(0) == 0)
    def _():
        ac_s[...] = _bn_affine(nodes_ref[...], gamma_ref[...], beta_ref[...])

    x = xs_ref[...] * ac_s[0:1, :] + ac_s[1:2, :]
    h1 = jnp.maximum(e_ref[...] * w1_ref[...] + b1_ref[...], 0.0)
    h2 = jnp.maximum(
        jnp.dot(h1.astype(jnp.bfloat16), w2_ref[...],
                preferred_element_type=jnp.float32) + b2_ref[...], 0.0)
    w = (jnp.dot(h2.astype(jnp.bfloat16), w3_ref[...],
                 preferred_element_type=jnp.float32) + b3_ref[...])
    xr = jnp.dot(x, r_ref[...], preferred_element_type=jnp.float32)
    msg_ref[...] = jnp.dot(xr * w, s_ref[...],
                           preferred_element_type=jnp.float32)


def _dense(e, xs, nodes, gamma2, beta2, w1, b1r, w2b, b2r, w3b, b3r, r, s):
    full = lambda shape: pl.BlockSpec(shape, lambda i: (0, 0))
    return pl.pallas_call(
        _dense_body,
        grid=(E // EB,),
        in_specs=[
            pl.BlockSpec((EB, 1), lambda i: (i, 0)),
            pl.BlockSpec((EB, D), lambda i: (i, 0)),
            full((N, D)), full((1, D)), full((1, D)),
            full((1, H)), full((1, H)),
            full((H, H)), full((1, H)),
            full((H, DD)), full((1, DD)),
            full((D, DD)), full((DD, D)),
        ],
        out_specs=pl.BlockSpec((EB, D), lambda i: (i, 0)),
        out_shape=jax.ShapeDtypeStruct((E, D), jnp.float32),
        scratch_shapes=[pltpu.VMEM((2, D), jnp.float32)],
    )(e, xs, nodes, gamma2, beta2, w1, b1r, w2b, b2r, w3b, b3r, r, s)


def _update_body(aggp_ref, cntp_ref, bias_ref, hin_ref, hout_ref):
    agg = aggp_ref[0, :N, :] + aggp_ref[1, :N, :]
    cnt = cntp_ref[0, :N, 0:1] + cntp_ref[1, :N, 0:1]
    denom = jnp.maximum(cnt, 1.0)
    hout_ref[...] = agg / denom + bias_ref[...] + hin_ref[...]


def _update(aggp, cntp, bias2, hin):
    return pl.pallas_call(
        _update_body,
        out_shape=jax.ShapeDtypeStruct((N, D), jnp.float32),
    )(aggp, cntp, bias2, hin)


def kernel(h, e, edge_index, W1, b1, W2, b2, W3, b3, bias, gamma, beta):
    src2 = edge_index[1].reshape(NW * NCH, CH)
    dst2 = edge_index[0].reshape(NW * NCH, CH)
    b1r = b1.reshape(1, H)
    b2r = b2.reshape(1, H)
    b3r = b3.reshape(1, DD)
    bias2 = bias.reshape(1, D)
    gamma2 = gamma.reshape(1, D)
    beta2 = beta.reshape(1, D)
    w2b = W2.astype(jnp.bfloat16)
    w3b = W3.astype(jnp.bfloat16)
    r = jnp.kron(jnp.eye(D, dtype=jnp.float32),
                 jnp.ones((1, D), jnp.float32))
    s = jnp.kron(jnp.ones((D, 1), jnp.float32),
                 jnp.eye(D, dtype=jnp.float32))

    xs1, cntp = _sc_gather_counts(h, src2, dst2)
    msg1 = _dense(e, xs1, h, gamma2, beta2,
                  W1, b1r, w2b, b2r, w3b, b3r, r, s)
    aggp1 = _sc_scatter(dst2, msg1)
    h2 = _update(aggp1, cntp, bias2, h)
    xs2 = _sc_gather(h2, src2)
    msg2 = _dense(e, xs2, h2, gamma2, beta2,
                  W1, b1r, w2b, b2r, w3b, b3r, r, s)
    aggp2 = _sc_scatter(dst2, msg2)
    return _update(aggp2, cntp, bias2, h2)


# speedup vs baseline: 1.8163x; 1.8163x over previous
import functools

import jax
import jax.numpy as jnp
from jax import lax
from jax.experimental import pallas as pl
from jax.experimental.pallas import tpu as pltpu
from jax.experimental.pallas import tpu_sc as plsc

N = 10000
E = 160000
D = 16
H = 128
DD = D * D

NC = 2
NS = 16
NW = NC * NS
EPW = E // NW
CH = 125
NCH = EPW // CH
NP = 10240
RPS = NP // NS
NPK = NP // 8
NK = N // 8

EB = 3200
GR = EB // 8
NB = E // EB
EK = E // 8

_MESH = plsc.VectorSubcoreMesh(core_axis_name="c", subcore_axis_name="s")
_SC_PARAMS = pltpu.CompilerParams(use_tc_tiling_on_sc=False)


def _fill_rows(ref, n, value):
    def body(i, carry):
        ref[i, :] = jnp.full((D,), value, jnp.float32)
        return carry
    lax.fori_loop(0, n, body, 0)


def _fire_drain(n, fire):
    def fire_body(j, carry):
        fire(j)
        return carry
    lax.fori_loop(0, n, fire_body, 0)

    def drain_body(j, carry):
        fire(0, wait=True)
        return carry
    lax.fori_loop(0, n, drain_body, 0)


@functools.partial(
    pl.kernel,
    out_type=(
        jax.ShapeDtypeStruct((E, D), jnp.float32),
        jax.ShapeDtypeStruct((NC, NP, D), jnp.float32),
    ),
    scratch_types=[
        pltpu.VMEM((NCH, CH), jnp.int32),
        pltpu.VMEM((NCH, CH), jnp.int32),
        pltpu.VMEM((EPW, D), jnp.float32),
        pltpu.VMEM((CH, D), jnp.float32),
        pltpu.VMEM((RPS, D), jnp.float32),
        pltpu.SemaphoreType.DMA,
        pltpu.SemaphoreType.DMA,
        pltpu.VMEM_SHARED((NP, D), jnp.float32),
    ],
    mesh=_MESH,
    compiler_params=_SC_PARAMS,
)
def _sc_gather_counts(x_hbm, src_hbm, dst_hbm, xs_out, cnt_out,
                      sidx, didx, rows, obuf, zbuf, gsem, csem, cnt_sh):
    c = lax.axis_index("c")
    s = lax.axis_index("s")
    wid = s * NC + c
    pltpu.sync_copy(src_hbm.at[pl.ds(wid * NCH, NCH)], sidx)
    pltpu.sync_copy(dst_hbm.at[pl.ds(wid * NCH, NCH)], didx)
    _fill_rows(obuf, CH, 1.0)
    _fill_rows(zbuf, RPS, 0.0)
    pltpu.sync_copy(zbuf, cnt_sh.at[pl.ds(s * RPS, RPS)])
    plsc.subcore_barrier()

    def gath(j, wait=False):
        d = pltpu.make_async_copy(x_hbm.at[sidx.at[j]],
                                  rows.at[pl.ds(j * CH, CH)], gsem)
        d.wait() if wait else d.start()
    _fire_drain(NCH, gath)

    def cadd(j, wait=False):
        d = pltpu.make_async_copy(obuf, cnt_sh.at[didx.at[j]], csem)
        d.wait() if wait else d.start(add=True)
    _fire_drain(NCH, cadd)

    pltpu.sync_copy(rows, xs_out.at[pl.ds(wid * EPW, EPW)])
    plsc.subcore_barrier()
    pltpu.sync_copy(cnt_sh.at[pl.ds(s * RPS, RPS)],
                    cnt_out.at[c].at[pl.ds(s * RPS, RPS)])


@functools.partial(
    pl.kernel,
    out_type=jax.ShapeDtypeStruct((E, D), jnp.float32),
    scratch_types=[
        pltpu.VMEM((NCH, CH), jnp.int32),
        pltpu.VMEM((EPW, D), jnp.float32),
        pltpu.SemaphoreType.DMA,
    ],
    mesh=_MESH,
    compiler_params=_SC_PARAMS,
)
def _sc_gather(x_hbm, src_hbm, xs_out, sidx, rows, gsem):
    c = lax.axis_index("c")
    s = lax.axis_index("s")
    wid = s * NC + c
    pltpu.sync_copy(src_hbm.at[pl.ds(wid * NCH, NCH)], sidx)

    def gath(j, wait=False):
        d = pltpu.make_async_copy(x_hbm.at[sidx.at[j]],
                                  rows.at[pl.ds(j * CH, CH)], gsem)
        d.wait() if wait else d.start()
    _fire_drain(NCH, gath)

    pltpu.sync_copy(rows, xs_out.at[pl.ds(wid * EPW, EPW)])


@functools.partial(
    pl.kernel,
    out_type=jax.ShapeDtypeStruct((NC, NP, D), jnp.float32),
    scratch_types=[
        pltpu.VMEM((NCH, CH), jnp.int32),
        pltpu.VMEM((EPW, D), jnp.float32),
        pltpu.VMEM((RPS, D), jnp.float32),
        pltpu.SemaphoreType.DMA,
        pltpu.VMEM_SHARED((NP, D), jnp.float32),
    ],
    mesh=_MESH,
    compiler_params=_SC_PARAMS,
)
def _sc_scatter(dst_hbm, msg_hbm, agg_out, didx, buf, zbuf, asem, agg_sh):
    c = lax.axis_index("c")
    s = lax.axis_index("s")
    wid = s * NC + c
    pltpu.sync_copy(dst_hbm.at[pl.ds(wid * NCH, NCH)], didx)
    pltpu.sync_copy(msg_hbm.at[pl.ds(wid * EPW, EPW)], buf)
    _fill_rows(zbuf, RPS, 0.0)
    pltpu.sync_copy(zbuf, agg_sh.at[pl.ds(s * RPS, RPS)])
    plsc.subcore_barrier()

    def sadd(j, wait=False):
        d = pltpu.make_async_copy(buf.at[pl.ds(j * CH, CH)],
                                  agg_sh.at[didx.at[j]], asem)
        d.wait() if wait else d.start(add=True)
    _fire_drain(NCH, sadd)

    plsc.subcore_barrier()
    pltpu.sync_copy(agg_sh.at[pl.ds(s * RPS, RPS)],
                    agg_out.at[c].at[pl.ds(s * RPS, RPS)])


def _dense_body(e_ref, xs_ref, nodes_ref, gamma_ref, beta_ref, fold_ref,
                w1_ref, b1_ref, w2_ref, b2_ref, w3_ref, b3_ref,
                r_ref, s_ref, msg_ref, ac_s):
    @pl.when(pl.program_id(0) == 0)
    def _():
        nod = nodes_ref[...]
        ssum = jnp.dot(jnp.sum(nod, axis=0, keepdims=True), fold_ref[...],
                       preferred_element_type=jnp.float32)
        ssq = jnp.dot(jnp.sum(nod * nod, axis=0, keepdims=True), fold_ref[...],
                      preferred_element_type=jnp.float32)
        mu = ssum / float(N)
        var = ssq / float(N) - mu * mu
        a = gamma_ref[...] / jnp.sqrt(var + 1e-5)
        ac_s[...] = jnp.concatenate([a, beta_ref[...] - mu * a], axis=0)

    a = ac_s[0:1, :]
    c = ac_s[1:2, :]
    for g in range(8):
        ecol = e_ref[:, g:g + 1]
        h1 = jnp.maximum(ecol * w1_ref[...] + b1_ref[...], 0.0)
        h2 = jnp.maximum(
            jnp.dot(h1.astype(jnp.bfloat16), w2_ref[...],
                    preferred_element_type=jnp.float32) + b2_ref[...], 0.0)
        w = (jnp.dot(h2.astype(jnp.bfloat16), w3_ref[...],
                     preferred_element_type=jnp.float32) + b3_ref[...])
        xg = xs_ref[:, g * D:(g + 1) * D]
        xn = xg * a + c
        xr = jnp.dot(xn, r_ref[...], preferred_element_type=jnp.float32)
        msg_ref[:, g * D:(g + 1) * D] = jnp.dot(
            xr * w, s_ref[...], preferred_element_type=jnp.float32)


def _dense(e_t, xs_p, nodes_p, gamma2, beta2, fold,
           w1, b1r, w2b, b2r, w3b, b3r, r, s):
    full = lambda shape: pl.BlockSpec(shape, lambda i: (0, 0))
    return pl.pallas_call(
        _dense_body,
        grid=(NB,),
        in_specs=[
            pl.BlockSpec((GR, 8), lambda i: (i, 0)),
            pl.BlockSpec((GR, 128), lambda i: (i, 0)),
            full((NPK, 128)), full((1, D)), full((1, D)), full((128, D)),
            full((1, H)), full((1, H)),
            full((H, H)), full((1, H)),
            full((H, DD)), full((1, DD)),
            full((D, DD)), full((DD, D)),
        ],
        out_specs=pl.BlockSpec((GR, 128), lambda i: (i, 0)),
        out_shape=jax.ShapeDtypeStruct((EK, 128), jnp.float32),
        scratch_shapes=[pltpu.VMEM((2, D), jnp.float32)],
    )(e_t, xs_p, nodes_p, gamma2, beta2, fold,
      w1, b1r, w2b, b2r, w3b, b3r, r, s)


def _update_body(aggp_ref, cntp_ref, bias_ref, hin_ref, hout_ref):
    agg = aggp_ref[0] + aggp_ref[1]
    cnt = cntp_ref[0] + cntp_ref[1]
    denom = jnp.maximum(cnt, 1.0)
    hnew = agg / denom + bias_ref[...] + hin_ref[...]
    rowid = lax.broadcasted_iota(jnp.int32, (NPK, 128), 0)
    hout_ref[...] = jnp.where(rowid < NK, hnew, 0.0)


def _update(aggp, cntp, biasp, hinp):
    return pl.pallas_call(
        _update_body,
        out_shape=jax.ShapeDtypeStruct((NPK, 128), jnp.float32),
    )(aggp, cntp, biasp, hinp)


def _slotize(v):
    return v.reshape(NB, 8, GR).transpose(0, 2, 1)


def kernel(h, e, edge_index, W1, b1, W2, b2, W3, b3, bias, gamma, beta):
    src2 = _slotize(edge_index[1]).reshape(NW * NCH, CH)
    dst2 = _slotize(edge_index[0]).reshape(NW * NCH, CH)
    e_t = _slotize(e.reshape(E)).reshape(EK, 8)
    hp = jnp.concatenate(
        [h.reshape(NK, 128),
         jnp.zeros((NPK - NK, 128), jnp.float32)], axis=0)
    b1r = b1.reshape(1, H)
    b2r = b2.reshape(1, H)
    b3r = b3.reshape(1, DD)
    biasp = jnp.tile(bias, 8).reshape(1, 128)
    gamma2 = gamma.reshape(1, D)
    beta2 = beta.reshape(1, D)
    w2b = W2.astype(jnp.bfloat16)
    w3b = W3.astype(jnp.bfloat16)
    eye = jnp.eye(D, dtype=jnp.float32)
    r = jnp.kron(eye, jnp.ones((1, D), jnp.float32))
    s = jnp.kron(jnp.ones((D, 1), jnp.float32), eye)
    fold = jnp.kron(jnp.ones((8, 1), jnp.float32), eye)

    xs1, cntp = _sc_gather_counts(hp.reshape(NP, D), src2, dst2)
    msg1 = _dense(e_t, xs1.reshape(EK, 128), hp, gamma2, beta2, fold,
                  W1, b1r, w2b, b2r, w3b, b3r, r, s)
    aggp1 = _sc_scatter(dst2, msg1.reshape(E, D))
    h2p = _update(aggp1.reshape(NC, NPK, 128), cntp.reshape(NC, NPK, 128),
                  biasp, hp)
    xs2 = _sc_gather(h2p.reshape(NP, D), src2)
    msg2 = _dense(e_t, xs2.reshape(EK, 128), h2p, gamma2, beta2, fold,
                  W1, b1r, w2b, b2r, w3b, b3r, r, s)
    aggp2 = _sc_scatter(dst2, msg2.reshape(E, D))
    h3p = _update(aggp2.reshape(NC, NPK, 128), cntp.reshape(NC, NPK, 128),
                  biasp, h2p)
    return h3p[:NK].reshape(N, D)


# speedup vs baseline: 1.9498x; 1.0735x over previous
import functools

import numpy as np

import jax
import jax.numpy as jnp
from jax import lax
from jax.experimental import pallas as pl
from jax.experimental.pallas import tpu as pltpu
from jax.experimental.pallas import tpu_sc as plsc

N = 10000
E = 160000
D = 16
H = 128
DD = D * D

NC = 2
NS = 16
NW = NC * NS
EPW = E // NW
CH = 125
NCH = EPW // CH
NP = 10240
RPS = NP // NS
NPK = NP // 8
NK = N // 8

EB = 3200
GR = EB // 8
NB = E // EB
EK = E // 8

_MESH = plsc.VectorSubcoreMesh(core_axis_name="c", subcore_axis_name="s")
_SC_PARAMS = pltpu.CompilerParams(use_tc_tiling_on_sc=False)


def _fill_rows(ref, n, value):
    def body(i, carry):
        ref[i, :] = jnp.full((D,), value, jnp.float32)
        return carry
    lax.fori_loop(0, n, body, 0)


def _fire_drain(n, fire):
    def fire_body(j, carry):
        fire(j)
        return carry
    lax.fori_loop(0, n, fire_body, 0)

    def drain_body(j, carry):
        fire(0, wait=True)
        return carry
    lax.fori_loop(0, n, drain_body, 0)


@functools.partial(
    pl.kernel,
    out_type=(
        jax.ShapeDtypeStruct((E, D), jnp.float32),
        jax.ShapeDtypeStruct((NC, NP, D), jnp.float32),
    ),
    scratch_types=[
        pltpu.VMEM((NCH, CH), jnp.int32),
        pltpu.VMEM((NCH, CH), jnp.int32),
        pltpu.VMEM((EPW, D), jnp.float32),
        pltpu.VMEM((CH, D), jnp.float32),
        pltpu.VMEM((RPS, D), jnp.float32),
        pltpu.SemaphoreType.DMA,
        pltpu.SemaphoreType.DMA,
        pltpu.VMEM_SHARED((NP, D), jnp.float32),
    ],
    mesh=_MESH,
    compiler_params=_SC_PARAMS,
)
def _sc_gather_counts(x_hbm, src_hbm, dst_hbm, xs_out, cnt_out,
                      sidx, didx, rows, obuf, zbuf, gsem, csem, cnt_sh):
    c = lax.axis_index("c")
    s = lax.axis_index("s")
    wid = s * NC + c
    pltpu.sync_copy(src_hbm.at[pl.ds(wid * NCH, NCH)], sidx)
    pltpu.sync_copy(dst_hbm.at[pl.ds(wid * NCH, NCH)], didx)
    _fill_rows(obuf, CH, 1.0)
    _fill_rows(zbuf, RPS, 0.0)
    pltpu.sync_copy(zbuf, cnt_sh.at[pl.ds(s * RPS, RPS)])
    plsc.subcore_barrier()

    def both(j, wait=False):
        dg = pltpu.make_async_copy(x_hbm.at[sidx.at[j]],
                                   rows.at[pl.ds(j * CH, CH)], gsem)
        dc = pltpu.make_async_copy(obuf, cnt_sh.at[didx.at[j]], csem)
        if wait:
            dg.wait()
            dc.wait()
        else:
            dg.start()
            dc.start(add=True)
    _fire_drain(NCH, both)

    pltpu.sync_copy(rows, xs_out.at[pl.ds(wid * EPW, EPW)])
    plsc.subcore_barrier()
    pltpu.sync_copy(cnt_sh.at[pl.ds(s * RPS, RPS)],
                    cnt_out.at[c].at[pl.ds(s * RPS, RPS)])


@functools.partial(
    pl.kernel,
    out_type=jax.ShapeDtypeStruct((E, D), jnp.float32),
    scratch_types=[
        pltpu.VMEM((NCH, CH), jnp.int32),
        pltpu.VMEM((EPW, D), jnp.float32),
        pltpu.SemaphoreType.DMA,
    ],
    mesh=_MESH,
    compiler_params=_SC_PARAMS,
)
def _sc_gather(x_hbm, src_hbm, xs_out, sidx, rows, gsem):
    c = lax.axis_index("c")
    s = lax.axis_index("s")
    wid = s * NC + c
    pltpu.sync_copy(src_hbm.at[pl.ds(wid * NCH, NCH)], sidx)

    def gath(j, wait=False):
        d = pltpu.make_async_copy(x_hbm.at[sidx.at[j]],
                                  rows.at[pl.ds(j * CH, CH)], gsem)
        d.wait() if wait else d.start()
    _fire_drain(NCH, gath)

    pltpu.sync_copy(rows, xs_out.at[pl.ds(wid * EPW, EPW)])


@functools.partial(
    pl.kernel,
    out_type=jax.ShapeDtypeStruct((NC, NP, D), jnp.float32),
    scratch_types=[
        pltpu.VMEM((NCH, CH), jnp.int32),
        pltpu.VMEM((EPW, D), jnp.float32),
        pltpu.VMEM((RPS, D), jnp.float32),
        pltpu.SemaphoreType.DMA,
        pltpu.VMEM_SHARED((NP, D), jnp.float32),
    ],
    mesh=_MESH,
    compiler_params=_SC_PARAMS,
)
def _sc_scatter(dst_hbm, msg_hbm, agg_out, didx, buf, zbuf, asem, agg_sh):
    c = lax.axis_index("c")
    s = lax.axis_index("s")
    wid = s * NC + c
    pltpu.sync_copy(dst_hbm.at[pl.ds(wid * NCH, NCH)], didx)
    pltpu.sync_copy(msg_hbm.at[pl.ds(wid * EPW, EPW)], buf)
    _fill_rows(zbuf, RPS, 0.0)
    pltpu.sync_copy(zbuf, agg_sh.at[pl.ds(s * RPS, RPS)])
    plsc.subcore_barrier()

    def sadd(j, wait=False):
        d = pltpu.make_async_copy(buf.at[pl.ds(j * CH, CH)],
                                  agg_sh.at[didx.at[j]], asem)
        d.wait() if wait else d.start(add=True)
    _fire_drain(NCH, sadd)

    plsc.subcore_barrier()
    pltpu.sync_copy(agg_sh.at[pl.ds(s * RPS, RPS)],
                    agg_out.at[c].at[pl.ds(s * RPS, RPS)])


def _dense_body(e_ref, xs_ref, nodes_ref, gamma_ref, beta_ref, fold_ref,
                w1_ref, b1_ref, w2_ref, b2_ref, w3_ref, b3_ref,
                r_ref, s_ref, msg_ref, ac_s):
    @pl.when(pl.program_id(0) == 0)
    def _():
        nod = nodes_ref[...]
        ssum = jnp.dot(jnp.sum(nod, axis=0, keepdims=True), fold_ref[...],
                       preferred_element_type=jnp.float32)
        ssq = jnp.dot(jnp.sum(nod * nod, axis=0, keepdims=True), fold_ref[...],
                      preferred_element_type=jnp.float32)
        mu = ssum / float(N)
        var = ssq / float(N) - mu * mu
        a = gamma_ref[...] / jnp.sqrt(var + 1e-5)
        ac_s[...] = jnp.concatenate([a, beta_ref[...] - mu * a], axis=0)

    a = ac_s[0:1, :]
    c = ac_s[1:2, :]
    for g in range(8):
        ecol = e_ref[:, g:g + 1]
        h1 = jnp.maximum(ecol * w1_ref[...] + b1_ref[...], 0.0)
        h2 = jnp.maximum(
            jnp.dot(h1.astype(jnp.bfloat16), w2_ref[...],
                    preferred_element_type=jnp.float32) + b2_ref[...], 0.0)
        w = (jnp.dot(h2.astype(jnp.bfloat16), w3_ref[...],
                     preferred_element_type=jnp.float32) + b3_ref[...])
        xg = xs_ref[:, g * D:(g + 1) * D]
        xn = xg * a + c
        xr = jnp.dot(xn, r_ref[...], preferred_element_type=jnp.float32)
        msg_ref[:, g * D:(g + 1) * D] = jnp.dot(
            xr * w, s_ref[...], preferred_element_type=jnp.float32)


def _dense(e_t, xs_p, nodes_p, gamma2, beta2, fold,
           w1, b1r, w2b, b2r, w3b, b3r, r, s):
    full = lambda shape: pl.BlockSpec(shape, lambda i: (0, 0))
    return pl.pallas_call(
        _dense_body,
        grid=(NB,),
        in_specs=[
            pl.BlockSpec((GR, 8), lambda i: (i, 0)),
            pl.BlockSpec((GR, 128), lambda i: (i, 0)),
            full((NPK, 128)), full((1, D)), full((1, D)), full((128, D)),
            full((1, H)), full((1, H)),
            full((H, H)), full((1, H)),
            full((H, DD)), full((1, DD)),
            full((D, DD)), full((DD, D)),
        ],
        out_specs=pl.BlockSpec((GR, 128), lambda i: (i, 0)),
        out_shape=jax.ShapeDtypeStruct((EK, 128), jnp.float32),
        scratch_shapes=[pltpu.VMEM((2, D), jnp.float32)],
    )(e_t, xs_p, nodes_p, gamma2, beta2, fold,
      w1, b1r, w2b, b2r, w3b, b3r, r, s)


def _update_body(aggp_ref, cntp_ref, bias_ref, hin_ref, hout_ref):
    agg = aggp_ref[0] + aggp_ref[1]
    cnt = cntp_ref[0] + cntp_ref[1]
    denom = jnp.maximum(cnt, 1.0)
    hnew = agg / denom + bias_ref[...] + hin_ref[...]
    rowid = lax.broadcasted_iota(jnp.int32, (NPK, 128), 0)
    hout_ref[...] = jnp.where(rowid < NK, hnew, 0.0)


def _update(aggp, cntp, biasp, hinp):
    return pl.pallas_call(
        _update_body,
        out_shape=jax.ShapeDtypeStruct((NPK, 128), jnp.float32),
    )(aggp, cntp, biasp, hinp)


_EYE = np.eye(D, dtype=np.float32)
_R = jnp.asarray(np.kron(_EYE, np.ones((1, D), np.float32)))
_S = jnp.asarray(np.kron(np.ones((D, 1), np.float32), _EYE))
_FOLD = jnp.asarray(np.kron(np.ones((8, 1), np.float32), _EYE))


def kernel(h, e, edge_index, W1, b1, W2, b2, W3, b3, bias, gamma, beta):
    src2 = edge_index[1].reshape(NW * NCH, CH)
    dst2 = edge_index[0].reshape(NW * NCH, CH)
    e_t = e.reshape(EK, 8)
    hp = jnp.concatenate(
        [h.reshape(NK, 128),
         jnp.zeros((NPK - NK, 128), jnp.float32)], axis=0)
    b1r = b1.reshape(1, H)
    b2r = b2.reshape(1, H)
    b3r = b3.reshape(1, DD)
    biasp = jnp.tile(bias, 8).reshape(1, 128)
    gamma2 = gamma.reshape(1, D)
    beta2 = beta.reshape(1, D)
    w2b = W2.astype(jnp.bfloat16)
    w3b = W3.astype(jnp.bfloat16)
    r, s, fold = _R, _S, _FOLD

    xs1, cntp = _sc_gather_counts(hp.reshape(NP, D), src2, dst2)
    msg1 = _dense(e_t, xs1.reshape(EK, 128), hp, gamma2, beta2, fold,
                  W1, b1r, w2b, b2r, w3b, b3r, r, s)
    aggp1 = _sc_scatter(dst2, msg1.reshape(E, D))
    h2p = _update(aggp1.reshape(NC, NPK, 128), cntp.reshape(NC, NPK, 128),
                  biasp, hp)
    xs2 = _sc_gather(h2p.reshape(NP, D), src2)
    msg2 = _dense(e_t, xs2.reshape(EK, 128), h2p, gamma2, beta2, fold,
                  W1, b1r, w2b, b2r, w3b, b3r, r, s)
    aggp2 = _sc_scatter(dst2, msg2.reshape(E, D))
    h3p = _update(aggp2.reshape(NC, NPK, 128), cntp.reshape(NC, NPK, 128),
                  biasp, h2p)
    return h3p[:NK].reshape(N, D)


# speedup vs baseline: 2.3687x; 1.2148x over previous
import functools

import numpy as np

import jax
import jax.numpy as jnp
from jax import lax
from jax.experimental import pallas as pl
from jax.experimental.pallas import tpu as pltpu
from jax.experimental.pallas import tpu_sc as plsc

N = 10000
E = 160000
D = 16
H = 128
DD = D * D

NC = 2
NS = 16
NW = NC * NS
EPW = E // NW
CH = 125
NCH = EPW // CH
NP = 10240
RPS = NP // NS
NPK = NP // 8
NK = N // 8

EB = 32000
GR = EB // 8
NB = E // EB
EK = E // 8

_MESH = plsc.VectorSubcoreMesh(core_axis_name="c", subcore_axis_name="s")
_SC_PARAMS = pltpu.CompilerParams(use_tc_tiling_on_sc=False)


def _fill_rows(ref, n, value):
    def body(i, carry):
        ref[i, :] = jnp.full((D,), value, jnp.float32)
        return carry
    lax.fori_loop(0, n, body, 0)


def _fire_drain(n, fire):
    def fire_body(j, carry):
        fire(j)
        return carry
    lax.fori_loop(0, n, fire_body, 0)

    def drain_body(j, carry):
        fire(0, wait=True)
        return carry
    lax.fori_loop(0, n, drain_body, 0)


@functools.partial(
    pl.kernel,
    out_type=(
        jax.ShapeDtypeStruct((E, D), jnp.float32),
        jax.ShapeDtypeStruct((NC, NP, D), jnp.float32),
    ),
    scratch_types=[
        pltpu.VMEM((NCH, CH), jnp.int32),
        pltpu.VMEM((NCH, CH), jnp.int32),
        pltpu.VMEM((EPW, D), jnp.float32),
        pltpu.VMEM((CH, D), jnp.float32),
        pltpu.VMEM((RPS, D), jnp.float32),
        pltpu.SemaphoreType.DMA,
        pltpu.SemaphoreType.DMA,
        pltpu.VMEM_SHARED((NP, D), jnp.float32),
    ],
    mesh=_MESH,
    compiler_params=_SC_PARAMS,
)
def _sc_gather_counts(x_hbm, src_hbm, dst_hbm, xs_out, cnt_out,
                      sidx, didx, rows, obuf, zbuf, gsem, csem, cnt_sh):
    c = lax.axis_index("c")
    s = lax.axis_index("s")
    wid = s * NC + c
    pltpu.sync_copy(src_hbm.at[pl.ds(wid * NCH, NCH)], sidx)
    pltpu.sync_copy(dst_hbm.at[pl.ds(wid * NCH, NCH)], didx)
    _fill_rows(obuf, CH, 1.0)
    _fill_rows(zbuf, RPS, 0.0)
    pltpu.sync_copy(zbuf, cnt_sh.at[pl.ds(s * RPS, RPS)])
    plsc.subcore_barrier()

    def both(j, wait=False):
        dg = pltpu.make_async_copy(x_hbm.at[sidx.at[j]],
                                   rows.at[pl.ds(j * CH, CH)], gsem)
        dc = pltpu.make_async_copy(obuf, cnt_sh.at[didx.at[j]], csem)
        if wait:
            dg.wait()
            dc.wait()
        else:
            dg.start()
            dc.start(add=True)
    _fire_drain(NCH, both)

    pltpu.sync_copy(rows, xs_out.at[pl.ds(wid * EPW, EPW)])
    plsc.subcore_barrier()
    pltpu.sync_copy(cnt_sh.at[pl.ds(s * RPS, RPS)],
                    cnt_out.at[c].at[pl.ds(s * RPS, RPS)])


@functools.partial(
    pl.kernel,
    out_type=jax.ShapeDtypeStruct((E, D), jnp.float32),
    scratch_types=[
        pltpu.VMEM((NCH, CH), jnp.int32),
        pltpu.VMEM((EPW, D), jnp.float32),
        pltpu.SemaphoreType.DMA,
    ],
    mesh=_MESH,
    compiler_params=_SC_PARAMS,
)
def _sc_gather(x_hbm, src_hbm, xs_out, sidx, rows, gsem):
    c = lax.axis_index("c")
    s = lax.axis_index("s")
    wid = s * NC + c
    pltpu.sync_copy(src_hbm.at[pl.ds(wid * NCH, NCH)], sidx)

    def gath(j, wait=False):
        d = pltpu.make_async_copy(x_hbm.at[sidx.at[j]],
                                  rows.at[pl.ds(j * CH, CH)], gsem)
        d.wait() if wait else d.start()
    _fire_drain(NCH, gath)

    pltpu.sync_copy(rows, xs_out.at[pl.ds(wid * EPW, EPW)])


@functools.partial(
    pl.kernel,
    out_type=jax.ShapeDtypeStruct((NC, NP, D), jnp.float32),
    scratch_types=[
        pltpu.VMEM((NCH, CH), jnp.int32),
        pltpu.VMEM((EPW, D), jnp.float32),
        pltpu.VMEM((RPS, D), jnp.float32),
        pltpu.SemaphoreType.DMA,
        pltpu.VMEM_SHARED((NP, D), jnp.float32),
    ],
    mesh=_MESH,
    compiler_params=_SC_PARAMS,
)
def _sc_scatter(dst_hbm, msg_hbm, agg_out, didx, buf, zbuf, asem, agg_sh):
    c = lax.axis_index("c")
    s = lax.axis_index("s")
    wid = s * NC + c
    pltpu.sync_copy(dst_hbm.at[pl.ds(wid * NCH, NCH)], didx)
    pltpu.sync_copy(msg_hbm.at[pl.ds(wid * EPW, EPW)], buf)
    _fill_rows(zbuf, RPS, 0.0)
    pltpu.sync_copy(zbuf, agg_sh.at[pl.ds(s * RPS, RPS)])
    plsc.subcore_barrier()

    def sadd(j, wait=False):
        d = pltpu.make_async_copy(buf.at[pl.ds(j * CH, CH)],
                                  agg_sh.at[didx.at[j]], asem)
        d.wait() if wait else d.start(add=True)
    _fire_drain(NCH, sadd)

    plsc.subcore_barrier()
    pltpu.sync_copy(agg_sh.at[pl.ds(s * RPS, RPS)],
                    agg_out.at[c].at[pl.ds(s * RPS, RPS)])


def _dense_body(e_ref, xs_ref, nodes_ref, gamma_ref, beta_ref, fold_ref,
                w1_ref, b1_ref, w2_ref, b2_ref, w3_ref, b3_ref,
                r_ref, s_ref, msg_ref, ac_s):
    @pl.when(pl.program_id(0) == 0)
    def _():
        nod = nodes_ref[...]
        ssum = jnp.dot(jnp.sum(nod, axis=0, keepdims=True), fold_ref[...],
                       preferred_element_type=jnp.float32)
        ssq = jnp.dot(jnp.sum(nod * nod, axis=0, keepdims=True), fold_ref[...],
                      preferred_element_type=jnp.float32)
        mu = ssum / float(N)
        var = ssq / float(N) - mu * mu
        a = gamma_ref[...] / jnp.sqrt(var + 1e-5)
        ac_s[...] = jnp.concatenate([a, beta_ref[...] - mu * a], axis=0)

    a = ac_s[0:1, :]
    c = ac_s[1:2, :]
    for g in range(8):
        ecol = e_ref[:, g:g + 1]
        h1 = jnp.maximum(ecol * w1_ref[...] + b1_ref[...], 0.0)
        h2 = jnp.maximum(
            jnp.dot(h1.astype(jnp.bfloat16), w2_ref[...],
                    preferred_element_type=jnp.float32) + b2_ref[...], 0.0)
        w = (jnp.dot(h2.astype(jnp.bfloat16), w3_ref[...],
                     preferred_element_type=jnp.float32) + b3_ref[...])
        xg = xs_ref[:, g * D:(g + 1) * D]
        xn = xg * a + c
        xr = jnp.dot(xn, r_ref[...], preferred_element_type=jnp.float32)
        msg_ref[:, g * D:(g + 1) * D] = jnp.dot(
            xr * w, s_ref[...], preferred_element_type=jnp.float32)


def _dense(e_t, xs_p, nodes_p, gamma2, beta2, fold,
           w1, b1r, w2b, b2r, w3b, b3r, r, s):
    full = lambda shape: pl.BlockSpec(shape, lambda i: (0, 0))
    return pl.pallas_call(
        _dense_body,
        grid=(NB,),
        in_specs=[
            pl.BlockSpec((GR, 8), lambda i: (i, 0)),
            pl.BlockSpec((GR, 128), lambda i: (i, 0)),
            full((NPK, 128)), full((1, D)), full((1, D)), full((128, D)),
            full((1, H)), full((1, H)),
            full((H, H)), full((1, H)),
            full((H, DD)), full((1, DD)),
            full((D, DD)), full((DD, D)),
        ],
        out_specs=pl.BlockSpec((GR, 128), lambda i: (i, 0)),
        out_shape=jax.ShapeDtypeStruct((EK, 128), jnp.float32),
        scratch_shapes=[pltpu.VMEM((2, D), jnp.float32)],
    )(e_t, xs_p, nodes_p, gamma2, beta2, fold,
      w1, b1r, w2b, b2r, w3b, b3r, r, s)


def _update_body(aggp_ref, cntp_ref, bias_ref, hin_ref, hout_ref):
    agg = aggp_ref[0] + aggp_ref[1]
    cnt = cntp_ref[0] + cntp_ref[1]
    denom = jnp.maximum(cnt, 1.0)
    hnew = agg / denom + bias_ref[...] + hin_ref[...]
    rowid = lax.broadcasted_iota(jnp.int32, (NPK, 128), 0)
    hout_ref[...] = jnp.where(rowid < NK, hnew, 0.0)


def _update(aggp, cntp, biasp, hinp):
    return pl.pallas_call(
        _update_body,
        out_shape=jax.ShapeDtypeStruct((NPK, 128), jnp.float32),
    )(aggp, cntp, biasp, hinp)


_EYE = np.eye(D, dtype=np.float32)
_R = np.kron(_EYE, np.ones((1, D), np.float32))
_S = np.kron(np.ones((D, 1), np.float32), _EYE)
_FOLD = np.kron(np.ones((8, 1), np.float32), _EYE)
_PERM = np.arange(DD).reshape(D, D).T.reshape(DD)
_S2 = np.kron(_EYE, np.ones((D, 1), np.float32))


def kernel(h, e, edge_index, W1, b1, W2, b2, W3, b3, bias, gamma, beta):
    src2 = edge_index[1].reshape(NW * NCH, CH)
    dst2 = edge_index[0].reshape(NW * NCH, CH)
    e_t = e.reshape(EK, 8)
    hp = jnp.concatenate(
        [h.reshape(NK, 128),
         jnp.zeros((NPK - NK, 128), jnp.float32)], axis=0)
    b1r = b1.reshape(1, H)
    b2r = b2.reshape(1, H)
    b3r = b3.reshape(1, DD)
    biasp = jnp.tile(bias, 8).reshape(1, 128)
    gamma2 = gamma.reshape(1, D)
    beta2 = beta.reshape(1, D)
    w2b = W2.astype(jnp.bfloat16)
    w3b = W3.astype(jnp.bfloat16)
    r = jnp.asarray(_R)
    s = jnp.asarray(_S)
    fold = jnp.asarray(_FOLD)

    xs1, cntp = _sc_gather_counts(hp.reshape(NP, D), src2, dst2)
    msg1 = _dense(e_t, xs1.reshape(EK, 128), hp, gamma2, beta2, fold,
                  W1, b1r, w2b, b2r, w3b, b3r, r, s)
    aggp1 = _sc_scatter(dst2, msg1.reshape(E, D))
    h2p = _update(aggp1.reshape(NC, NPK, 128), cntp.reshape(NC, NPK, 128),
                  biasp, hp)
    xs2 = _sc_gather(h2p.reshape(NP, D), src2)
    msg2 = _dense(e_t, xs2.reshape(EK, 128), h2p, gamma2, beta2, fold,
                  W1, b1r, w2b, b2r, w3b, b3r, r, s)
    aggp2 = _sc_scatter(dst2, msg2.reshape(E, D))
    h3p = _update(aggp2.reshape(NC, NPK, 128), cntp.reshape(NC, NPK, 128),
                  biasp, h2p)
    return h3p[:NK].reshape(N, D)


# speedup vs baseline: 2.4394x; 1.0298x over previous
import functools

import numpy as np

import jax
import jax.numpy as jnp
from jax import lax
from jax.experimental import pallas as pl
from jax.experimental.pallas import tpu as pltpu
from jax.experimental.pallas import tpu_sc as plsc

N = 10000
E = 160000
D = 16
H = 128
DD = D * D

NC = 2
NS = 16
NW = NC * NS
EPW = E // NW
CH = 125
NCH = EPW // CH
NP = 10240
RPS = NP // NS
NPK = NP // 8
NK = N // 8

EB = 32000
GR = EB // 8
NB = E // EB
EK = E // 8

_MESH = plsc.VectorSubcoreMesh(core_axis_name="c", subcore_axis_name="s")
_SC_PARAMS = pltpu.CompilerParams(use_tc_tiling_on_sc=False)


def _fill_rows(ref, n, value):
    def body(i, carry):
        ref[i, :] = jnp.full((D,), value, jnp.float32)
        return carry
    lax.fori_loop(0, n, body, 0)


def _fire_drain(n, fire):
    def fire_body(j, carry):
        fire(j)
        return carry
    lax.fori_loop(0, n, fire_body, 0)

    def drain_body(j, carry):
        fire(0, wait=True)
        return carry
    lax.fori_loop(0, n, drain_body, 0)


@functools.partial(
    pl.kernel,
    out_type=jax.ShapeDtypeStruct((NC, NP, D), jnp.float32),
    scratch_types=[
        pltpu.VMEM((NCH, CH), jnp.int32),
        pltpu.VMEM((CH, D), jnp.float32),
        pltpu.VMEM((RPS, D), jnp.float32),
        pltpu.SemaphoreType.DMA,
        pltpu.VMEM_SHARED((NP, D), jnp.float32),
    ],
    mesh=_MESH,
    compiler_params=_SC_PARAMS,
)
def _sc_counts(dst_hbm, cnt_out, didx, obuf, zbuf, csem, cnt_sh):
    c = lax.axis_index("c")
    s = lax.axis_index("s")
    wid = s * NC + c
    pltpu.sync_copy(dst_hbm.at[pl.ds(wid * NCH, NCH)], didx)
    _fill_rows(obuf, CH, 1.0)
    _fill_rows(zbuf, RPS, 0.0)
    pltpu.sync_copy(zbuf, cnt_sh.at[pl.ds(s * RPS, RPS)])
    plsc.subcore_barrier()

    def cadd(j, wait=False):
        d = pltpu.make_async_copy(obuf, cnt_sh.at[didx.at[j]], csem)
        d.wait() if wait else d.start(add=True)
    _fire_drain(NCH, cadd)

    plsc.subcore_barrier()
    pltpu.sync_copy(cnt_sh.at[pl.ds(s * RPS, RPS)],
                    cnt_out.at[c].at[pl.ds(s * RPS, RPS)])


@functools.partial(
    pl.kernel,
    out_type=jax.ShapeDtypeStruct((E, D), jnp.float32),
    scratch_types=[
        pltpu.VMEM((NCH, CH), jnp.int32),
        pltpu.VMEM((EPW, D), jnp.float32),
        pltpu.SemaphoreType.DMA,
    ],
    mesh=_MESH,
    compiler_params=_SC_PARAMS,
)
def _sc_gather(x_hbm, src_hbm, xs_out, sidx, rows, gsem):
    c = lax.axis_index("c")
    s = lax.axis_index("s")
    wid = s * NC + c
    pltpu.sync_copy(src_hbm.at[pl.ds(wid * NCH, NCH)], sidx)

    def gath(j, wait=False):
        d = pltpu.make_async_copy(x_hbm.at[sidx.at[j]],
                                  rows.at[pl.ds(j * CH, CH)], gsem)
        d.wait() if wait else d.start()
    _fire_drain(NCH, gath)

    pltpu.sync_copy(rows, xs_out.at[pl.ds(wid * EPW, EPW)])


@functools.partial(
    pl.kernel,
    out_type=jax.ShapeDtypeStruct((NC, NP, D), jnp.float32),
    scratch_types=[
        pltpu.VMEM((NCH, CH), jnp.int32),
        pltpu.VMEM((EPW, D), jnp.float32),
        pltpu.VMEM((RPS, D), jnp.float32),
        pltpu.SemaphoreType.DMA,
        pltpu.VMEM_SHARED((NP, D), jnp.float32),
    ],
    mesh=_MESH,
    compiler_params=_SC_PARAMS,
)
def _sc_scatter(dst_hbm, msg_hbm, agg_out, didx, buf, zbuf, asem, agg_sh):
    c = lax.axis_index("c")
    s = lax.axis_index("s")
    wid = s * NC + c
    pltpu.sync_copy(dst_hbm.at[pl.ds(wid * NCH, NCH)], didx)
    pltpu.sync_copy(msg_hbm.at[pl.ds(wid * EPW, EPW)], buf)
    _fill_rows(zbuf, RPS, 0.0)
    pltpu.sync_copy(zbuf, agg_sh.at[pl.ds(s * RPS, RPS)])
    plsc.subcore_barrier()

    def sadd(j, wait=False):
        d = pltpu.make_async_copy(buf.at[pl.ds(j * CH, CH)],
                                  agg_sh.at[didx.at[j]], asem)
        d.wait() if wait else d.start(add=True)
    _fire_drain(NCH, sadd)

    plsc.subcore_barrier()
    pltpu.sync_copy(agg_sh.at[pl.ds(s * RPS, RPS)],
                    agg_out.at[c].at[pl.ds(s * RPS, RPS)])


def _dense_body(e_ref, xs_ref, nodes_ref, gamma_ref, beta_ref, fold_ref,
                w1_ref, b1_ref, w2_ref, b2_ref, w3_ref, b3_ref,
                r_ref, s_ref, msg_ref, ac_s):
    @pl.when(pl.program_id(0) == 0)
    def _():
        nod = nodes_ref[...]
        ssum = jnp.dot(jnp.sum(nod, axis=0, keepdims=True), fold_ref[...],
                       preferred_element_type=jnp.float32)
        ssq = jnp.dot(jnp.sum(nod * nod, axis=0, keepdims=True), fold_ref[...],
                      preferred_element_type=jnp.float32)
        mu = ssum / float(N)
        var = ssq / float(N) - mu * mu
        a = gamma_ref[...] / jnp.sqrt(var + 1e-5)
        ac_s[...] = jnp.concatenate([a, beta_ref[...] - mu * a], axis=0)

    a = ac_s[0:1, :]
    c = ac_s[1:2, :]
    for g in range(8):
        ecol = e_ref[:, g:g + 1]
        h1 = jnp.maximum(ecol * w1_ref[...] + b1_ref[...], 0.0)
        h2 = jnp.maximum(
            jnp.dot(h1.astype(jnp.bfloat16), w2_ref[...],
                    preferred_element_type=jnp.float32) + b2_ref[...], 0.0)
        w = (jnp.dot(h2.astype(jnp.bfloat16), w3_ref[...],
                     preferred_element_type=jnp.float32) + b3_ref[...])
        xg = xs_ref[:, g * D:(g + 1) * D]
        xn = xg * a + c
        xr = jnp.dot(xn, r_ref[...], preferred_element_type=jnp.float32)
        msg_ref[:, g * D:(g + 1) * D] = jnp.dot(
            xr * w, s_ref[...], preferred_element_type=jnp.float32)


def _dense(e_t, xs_p, nodes_p, gamma2, beta2, fold,
           w1, b1r, w2b, b2r, w3b, b3r, r, s):
    full = lambda shape: pl.BlockSpec(shape, lambda i: (0, 0))
    return pl.pallas_call(
        _dense_body,
        grid=(NB,),
        in_specs=[
            pl.BlockSpec((GR, 8), lambda i: (i, 0)),
            pl.BlockSpec((GR, 128), lambda i: (i, 0)),
            full((NPK, 128)), full((1, D)), full((1, D)), full((128, D)),
            full((1, H)), full((1, H)),
            full((H, H)), full((1, H)),
            full((H, DD)), full((1, DD)),
            full((D, DD)), full((DD, D)),
        ],
        out_specs=pl.BlockSpec((GR, 128), lambda i: (i, 0)),
        out_shape=jax.ShapeDtypeStruct((EK, 128), jnp.float32),
        scratch_shapes=[pltpu.VMEM((2, D), jnp.float32)],
    )(e_t, xs_p, nodes_p, gamma2, beta2, fold,
      w1, b1r, w2b, b2r, w3b, b3r, r, s)


def _update_body(aggp_ref, cntp_ref, bias_ref, hin_ref, hout_ref):
    agg = aggp_ref[0] + aggp_ref[1]
    cnt = cntp_ref[0] + cntp_ref[1]
    denom = jnp.maximum(cnt, 1.0)
    hnew = agg / denom + bias_ref[...] + hin_ref[...]
    rowid = lax.broadcasted_iota(jnp.int32, (NPK, 128), 0)
    hout_ref[...] = jnp.where(rowid < NK, hnew, 0.0)


def _update(aggp, cntp, biasp, hinp):
    return pl.pallas_call(
        _update_body,
        out_shape=jax.ShapeDtypeStruct((NPK, 128), jnp.float32),
    )(aggp, cntp, biasp, hinp)


_EYE = np.eye(D, dtype=np.float32)
_R = np.kron(_EYE, np.ones((1, D), np.float32))
_S = np.kron(np.ones((D, 1), np.float32), _EYE)
_FOLD = np.kron(np.ones((8, 1), np.float32), _EYE)
_PERM = np.arange(DD).reshape(D, D).T.reshape(DD)
_S2 = np.kron(_EYE, np.ones((D, 1), np.float32))


def kernel(h, e, edge_index, W1, b1, W2, b2, W3, b3, bias, gamma, beta):
    src2 = edge_index[1].reshape(NW * NCH, CH)
    dst2 = edge_index[0].reshape(NW * NCH, CH)
    e_t = e.reshape(EK, 8)
    hp = jnp.concatenate(
        [h.reshape(NK, 128),
         jnp.zeros((NPK - NK, 128), jnp.float32)], axis=0)
    b1r = b1.reshape(1, H)
    b2r = b2.reshape(1, H)
    b3r = b3.reshape(1, DD)
    biasp = jnp.tile(bias, 8).reshape(1, 128)
    gamma2 = gamma.reshape(1, D)
    beta2 = beta.reshape(1, D)
    w2b = W2.astype(jnp.bfloat16)
    w3b = W3.astype(jnp.bfloat16)
    r = jnp.asarray(_R)
    s = jnp.asarray(_S)
    fold = jnp.asarray(_FOLD)

    xs1 = _sc_gather(hp.reshape(NP, D), src2)
    cntp = _sc_counts(dst2)
    msg1 = _dense(e_t, xs1.reshape(EK, 128), hp, gamma2, beta2, fold,
                  W1, b1r, w2b, b2r, w3b, b3r, r, s)
    aggp1 = _sc_scatter(dst2, msg1.reshape(E, D))
    h2p = _update(aggp1.reshape(NC, NPK, 128), cntp.reshape(NC, NPK, 128),
                  biasp, hp)
    xs2 = _sc_gather(h2p.reshape(NP, D), src2)
    msg2 = _dense(e_t, xs2.reshape(EK, 128), h2p, gamma2, beta2, fold,
                  W1, b1r, w2b, b2r, w3b, b3r, r, s)
    aggp2 = _sc_scatter(dst2, msg2.reshape(E, D))
    h3p = _update(aggp2.reshape(NC, NPK, 128), cntp.reshape(NC, NPK, 128),
                  biasp, h2p)
    return h3p[:NK].reshape(N, D)
